# baseline (speedup 1.0000x reference)
import functools

import jax
import jax.numpy as jnp
from jax import lax
from jax.experimental import pallas as pl
from jax.experimental.pallas import tpu as pltpu
from jax.experimental.pallas import tpu_sc as plsc

N = 10000
E = 320000
F = 128
R = 8

NC = 2
NS = 16
NW = NC * NS
C = 80
K = 125
NP = 10240
RPT = NP // NS


def _matmul_body(h_ref, w_ref, out_ref):
    for r in range(R):
        out_ref[r] = jnp.dot(h_ref[...], w_ref[r],
                             preferred_element_type=jnp.float32)


def _add_body(a_ref, b_ref, out_ref):
    out_ref[...] = a_ref[...] + b_ref[...]


def _edge_body(hw_hbm, pk_hbm, out_hbm, pkb, idxv, rows, acc, isem, gsem,
               ssem):
    cid = lax.axis_index("c")
    sid = lax.axis_index("s")
    wid = sid * NC + cid
    cbase = wid * K

    zero = jnp.zeros((16,), jnp.float32)

    def zrow(i, _):
        for g in range(F // 16):
            rows[0, i, pl.ds(g * 16, 16)] = zero
        return 0

    lax.fori_loop(0, C, zrow, 0)
    abase = sid * RPT
    for j in range(RPT // C):
        pltpu.sync_copy(rows.at[0], acc.at[pl.ds(abase + j * C, C)])
    plsc.subcore_barrier()

    def fire_idx(c, slot, sp):
        pltpu.async_copy(pk_hbm.at[cbase + c], pkb.at[slot], isem.at[sp])

    def wait_idx(c, slot, sp):
        pltpu.make_async_copy(pk_hbm.at[cbase + c], pkb.at[slot],
                              isem.at[sp]).wait()

    def compute_idx(slot, p):
        for j in range(C // 16):
            s = pkb[slot, 0, pl.ds(j * 16, 16)]
            r = pkb[slot, 1, pl.ds(j * 16, 16)]
            idxv[p, pl.ds(j * 16, 16)] = r * N + s

    def fire_gather(p, s):
        pltpu.async_copy(hw_hbm.at[idxv.at[p]], rows.at[s], gsem.at[p])

    def wait_gather(p, s):
        pltpu.make_async_copy(hw_hbm.at[idxv.at[p]], rows.at[s],
                              gsem.at[p]).wait()

    def fire_scatter(s):
        pltpu.async_copy(rows.at[s], acc.at[pkb.at[s, 2]], ssem.at[s],
                         add=True)

    def wait_scatter(s):
        pltpu.make_async_copy(rows.at[s], acc.at[pkb.at[s, 2]],
                              ssem.at[s]).wait()

    fire_idx(0, 0, 0)
    fire_idx(1, 1, 1)
    wait_idx(0, 0, 0)
    compute_idx(0, 0)
    fire_gather(0, 0)

    def chunk(k, _):
        p = k & 1
        q = 1 - p
        m = k & 3
        m1 = (k + 1) & 3
        m2 = (k + 2) & 3

        @pl.when(jnp.logical_and(k >= 2, k + 2 < K))
        def _():
            wait_scatter(m2)

        @pl.when(k + 2 < K)
        def _():
            fire_idx(k + 2, m2, p)

        @pl.when(k + 1 < K)
        def _():
            wait_idx(k + 1, m1, q)
            compute_idx(m1, q)
            fire_gather(q, m1)

        wait_gather(p, m)

        for j in range(C // 16):
            nv = lax.bitcast_convert_type(pkb[m, 3, pl.ds(j * 16, 16)],
                                          jnp.float32)
            for i in range(16):
                e = j * 16 + i
                sc = nv[i]
                for g in range(F // 16):
                    rows[m, e, pl.ds(g * 16, 16)] = (
                        rows[m, e, pl.ds(g * 16, 16)] * sc)

        fire_scatter(m)
        return 0

    lax.fori_loop(0, K, chunk, 0)
    for c in range(K - 4, K):
        wait_scatter(c & 3)
    plsc.subcore_barrier()

    obase = cid * NP + abase
    pltpu.sync_copy(acc.at[pl.ds(abase, RPT)], out_hbm.at[pl.ds(obase, RPT)])


_edge_kernel = functools.partial(
    pl.kernel,
    out_type=jax.ShapeDtypeStruct((NC * NP, F), jnp.float32),
    mesh=plsc.VectorSubcoreMesh(core_axis_name="c", subcore_axis_name="s"),
    scratch_types=[
        pltpu.VMEM((4, 4, C), jnp.int32),
        pltpu.VMEM((2, C), jnp.int32),
        pltpu.VMEM((4, C, F), jnp.float32),
        pltpu.VMEM_SHARED((NP, F), jnp.float32),
        pltpu.SemaphoreType.DMA((2,)),
        pltpu.SemaphoreType.DMA((2,)),
        pltpu.SemaphoreType.DMA((4,)),
    ],
)(_edge_body)


def kernel(h, edge_index, rel_type, norm, weight):
    hw = pl.pallas_call(
        _matmul_body,
        grid=(25,),
        in_specs=[
            pl.BlockSpec((400, F), lambda i: (i, 0)),
            pl.BlockSpec((R, F, F), lambda i: (0, 0, 0)),
        ],
        out_specs=pl.BlockSpec((R, 400, F), lambda i: (0, i, 0)),
        out_shape=jax.ShapeDtypeStruct((R, N, F), jnp.float32),
    )(h, weight)
    hw = hw.reshape(R * N, F)

    srcp = edge_index[0].reshape(NW * K, C)
    relp = rel_type.reshape(NW * K, C)
    dstp = edge_index[1].reshape(NW * K, C)
    nrmp = lax.bitcast_convert_type(norm.reshape(E), jnp.int32
                                    ).reshape(NW * K, C)
    packed = jnp.stack([srcp, relp, dstp, nrmp], axis=1)

    partial = _edge_kernel(hw, packed)

    BS = 1024
    out = pl.pallas_call(
        _add_body,
        grid=(NP // BS,),
        in_specs=[
            pl.BlockSpec((BS, F), lambda i: (i, 0)),
            pl.BlockSpec((BS, F), lambda i: (i + NP // BS, 0)),
        ],
        out_specs=pl.BlockSpec((BS, F), lambda i: (i, 0)),
        out_shape=jax.ShapeDtypeStruct((NP, F), jnp.float32),
    )(partial, partial)
    return out[:N]


# speedup vs baseline: 1.0991x; 1.0991x over previous
import functools

import jax
import jax.numpy as jnp
from jax import lax
from jax.experimental import pallas as pl
from jax.experimental.pallas import tpu as pltpu
from jax.experimental.pallas import tpu_sc as plsc

N = 10000
E = 320000
F = 128
R = 8

NC = 2
NS = 16
NW = NC * NS
C = 80
K = 125
NP = 10240
RPT = NP // NS


def _matmul_body(h_ref, w_ref, out_ref):
    for r in range(R):
        out_ref[r] = jnp.dot(h_ref[...], w_ref[r],
                             preferred_element_type=jnp.float32)


def _add_body(a_ref, b_ref, out_ref):
    out_ref[...] = a_ref[...] + b_ref[...]


def _edge_body(hw_hbm, src_hbm, rel_hbm, dst_hbm, nrm_hbm, out_hbm,
               srcb, relb, dstb, nrmb, idxv, rows, acc, isem, gsem, ssem):
    cid = lax.axis_index("c")
    sid = lax.axis_index("s")
    wid = sid * NC + cid
    cbase = wid * K

    zero = jnp.zeros((16,), jnp.float32)

    def zrow(i, _):
        for g in range(F // 16):
            rows[0, i, pl.ds(g * 16, 16)] = zero
        return 0

    lax.fori_loop(0, C, zrow, 0)
    abase = sid * RPT
    for j in range(RPT // C):
        pltpu.sync_copy(rows.at[0], acc.at[pl.ds(abase + j * C, C)])
    plsc.subcore_barrier()

    def fire_idx(c, slot, sp):
        base = (cbase + c) * C
        pltpu.async_copy(src_hbm.at[pl.ds(base, C)], srcb.at[slot],
                         isem.at[sp])
        pltpu.async_copy(rel_hbm.at[pl.ds(base, C)], relb.at[slot],
                         isem.at[sp])
        pltpu.async_copy(dst_hbm.at[pl.ds(base, C)], dstb.at[slot],
                         isem.at[sp])
        pltpu.async_copy(nrm_hbm.at[pl.ds(base, C)], nrmb.at[slot],
                         isem.at[sp])

    def wait_idx(c, slot, sp):
        base = (cbase + c) * C
        pltpu.make_async_copy(src_hbm.at[pl.ds(base, C)], srcb.at[slot],
                              isem.at[sp]).wait()
        pltpu.make_async_copy(rel_hbm.at[pl.ds(base, C)], relb.at[slot],
                              isem.at[sp]).wait()
        pltpu.make_async_copy(dst_hbm.at[pl.ds(base, C)], dstb.at[slot],
                              isem.at[sp]).wait()
        pltpu.make_async_copy(nrm_hbm.at[pl.ds(base, C)], nrmb.at[slot],
                              isem.at[sp]).wait()

    def compute_idx(slot, p):
        for j in range(C // 16):
            s = srcb[slot, pl.ds(j * 16, 16)]
            r = relb[slot, pl.ds(j * 16, 16)]
            idxv[p, pl.ds(j * 16, 16)] = r * N + s

    def fire_gather(p, s):
        pltpu.async_copy(hw_hbm.at[idxv.at[p]], rows.at[s], gsem.at[p])

    def wait_gather(p, s):
        pltpu.make_async_copy(hw_hbm.at[idxv.at[p]], rows.at[s],
                              gsem.at[p]).wait()

    def fire_scatter(s):
        pltpu.async_copy(rows.at[s], acc.at[dstb.at[s]], ssem.at[s],
                         add=True)

    def wait_scatter(s):
        pltpu.make_async_copy(rows.at[s], acc.at[dstb.at[s]],
                              ssem.at[s]).wait()

    fire_idx(0, 0, 0)
    fire_idx(1, 1, 1)
    wait_idx(0, 0, 0)
    compute_idx(0, 0)
    fire_gather(0, 0)

    def chunk(k, _):
        p = k & 1
        q = 1 - p
        m = k & 3
        m1 = (k + 1) & 3
        m2 = (k + 2) & 3

        @pl.when(jnp.logical_and(k >= 2, k + 2 < K))
        def _():
            wait_scatter(m2)

        @pl.when(k + 2 < K)
        def _():
            fire_idx(k + 2, m2, p)

        @pl.when(k + 1 < K)
        def _():
            wait_idx(k + 1, m1, q)
            compute_idx(m1, q)
            fire_gather(q, m1)

        wait_gather(p, m)

        for j in range(C // 16):
            nv = nrmb[m, pl.ds(j * 16, 16)]
            for i in range(16):
                e = j * 16 + i
                sc = nv[i]
                for g in range(F // 16):
                    rows[m, e, pl.ds(g * 16, 16)] = (
                        rows[m, e, pl.ds(g * 16, 16)] * sc)

        fire_scatter(m)
        return 0

    lax.fori_loop(0, K, chunk, 0)
    for c in range(K - 4, K):
        wait_scatter(c & 3)
    plsc.subcore_barrier()

    obase = cid * NP + abase
    pltpu.sync_copy(acc.at[pl.ds(abase, RPT)], out_hbm.at[pl.ds(obase, RPT)])


_edge_kernel = functools.partial(
    pl.kernel,
    out_type=jax.ShapeDtypeStruct((NC * NP, F), jnp.float32),
    mesh=plsc.VectorSubcoreMesh(core_axis_name="c", subcore_axis_name="s"),
    scratch_types=[
        pltpu.VMEM((4, C), jnp.int32),
        pltpu.VMEM((4, C), jnp.int32),
        pltpu.VMEM((4, C), jnp.int32),
        pltpu.VMEM((4, C), jnp.float32),
        pltpu.VMEM((2, C), jnp.int32),
        pltpu.VMEM((4, C, F), jnp.float32),
        pltpu.VMEM_SHARED((NP, F), jnp.float32),
        pltpu.SemaphoreType.DMA((2,)),
        pltpu.SemaphoreType.DMA((2,)),
        pltpu.SemaphoreType.DMA((4,)),
    ],
)(_edge_body)


def kernel(h, edge_index, rel_type, norm, weight):
    hw = pl.pallas_call(
        _matmul_body,
        grid=(25,),
        in_specs=[
            pl.BlockSpec((400, F), lambda i: (i, 0)),
            pl.BlockSpec((R, F, F), lambda i: (0, 0, 0)),
        ],
        out_specs=pl.BlockSpec((R, 400, F), lambda i: (0, i, 0)),
        out_shape=jax.ShapeDtypeStruct((R, N, F), jnp.float32),
    )(h, weight)
    hw = hw.reshape(R * N, F)

    partial = _edge_kernel(hw, edge_index[0], rel_type, edge_index[1],
                           norm.reshape(E))

    BS = 1024
    out = pl.pallas_call(
        _add_body,
        grid=(NP // BS,),
        in_specs=[
            pl.BlockSpec((BS, F), lambda i: (i, 0)),
            pl.BlockSpec((BS, F), lambda i: (i + NP // BS, 0)),
        ],
        out_specs=pl.BlockSpec((BS, F), lambda i: (i, 0)),
        out_shape=jax.ShapeDtypeStruct((NP, F), jnp.float32),
    )(partial, partial)
    return out[:N]
